# baseline (speedup 1.0000x reference)
import jax
import jax.numpy as jnp
from jax.experimental import pallas as pl
from jax.experimental.pallas import tpu as pltpu


def _fused_mse_kernel(x_ref, w_ref, b_ref, gt_ref, pred_ref, lpart_ref):
    c, h, w_sp = x_ref.shape[1:]
    k = w_ref.shape[0]
    x = x_ref[0].reshape(c, h * w_sp)
    pred = jax.lax.dot_general(
        w_ref[...], x, (((1,), (0,)), ((), ())),
        preferred_element_type=jnp.float32)
    pred = pred + b_ref[...]
    pred4 = pred.reshape(k, h, w_sp)
    pred_ref[0] = pred4
    d = pred4 - gt_ref[0]
    lpart_ref[...] = jnp.broadcast_to(jnp.sum(d * d), lpart_ref.shape)


def kernel(imgs, heatmaps, extra, wT, b):
    n, c, h, w = imgs.shape
    k = wT.shape[0]
    pred, lpart = pl.pallas_call(
        _fused_mse_kernel,
        out_shape=(jax.ShapeDtypeStruct((n, k, h, w), jnp.float32),
                   jax.ShapeDtypeStruct((n, 1, 128), jnp.float32)),
        grid=(n,),
        in_specs=[
            pl.BlockSpec((1, c, h, w), lambda i: (i, 0, 0, 0)),
            pl.BlockSpec((k, c), lambda i: (0, 0)),
            pl.BlockSpec((k, 1), lambda i: (0, 0)),
            pl.BlockSpec((1, k, h, w), lambda i: (i, 0, 0, 0)),
        ],
        out_specs=(
            pl.BlockSpec((1, k, h, w), lambda i: (i, 0, 0, 0)),
            pl.BlockSpec((1, 1, 128), lambda i: (i, 0, 0)),
        ),
        compiler_params=pltpu.CompilerParams(
            dimension_semantics=("parallel",)),
    )(imgs, wT, b, heatmaps)
    loss = jnp.sum(lpart[:, 0, 0]) * (1.0 / float(heatmaps.size))
    return [pred, loss]


# speedup vs baseline: 1.9085x; 1.9085x over previous
import functools

import jax
import jax.numpy as jnp
from jax.experimental import pallas as pl
from jax.experimental.pallas import tpu as pltpu


def _fused_mse_kernel(x_ref, w_ref, b_ref, gt_ref, pred_ref, lpart_ref, *,
                      hw_valid, padded):
    pred = jax.lax.dot_general(
        w_ref[...], x_ref[0], (((1,), (0,)), ((), ())),
        preferred_element_type=jnp.float32)
    pred = pred + b_ref[...]
    pred_ref[0] = pred.astype(pred_ref.dtype)
    d = pred - gt_ref[0].astype(jnp.float32)
    sq = d * d
    if padded:
        pos = jax.lax.broadcasted_iota(jnp.int32, sq.shape, 1)
        sq = jnp.where(pos < hw_valid, sq, 0.0)
    lpart_ref[...] = jnp.broadcast_to(jnp.sum(sq), lpart_ref.shape)


def _fused_call(x, wT, b, gt, hw):
    n, cin, hwp = x.shape
    k = wT.shape[0]
    kern = functools.partial(_fused_mse_kernel, hw_valid=hw,
                             padded=(hwp != hw))
    return pl.pallas_call(
        kern,
        out_shape=(jax.ShapeDtypeStruct((n, k, hwp), jnp.bfloat16),
                   jax.ShapeDtypeStruct((n, 1, 128), jnp.float32)),
        grid=(n,),
        in_specs=[
            pl.BlockSpec((1, cin, hwp), lambda i: (i, 0, 0)),
            pl.BlockSpec((k, cin), lambda i: (0, 0)),
            pl.BlockSpec((k, 1), lambda i: (0, 0)),
            pl.BlockSpec((1, k, hwp), lambda i: (i, 0, 0)),
        ],
        out_specs=(
            pl.BlockSpec((1, k, hwp), lambda i: (i, 0, 0)),
            pl.BlockSpec((1, 1, 128), lambda i: (i, 0, 0)),
        ),
        compiler_params=pltpu.CompilerParams(
            dimension_semantics=("parallel",)),
    )(x, wT, b, gt)


def kernel(imgs, heatmaps, extra, wT, b):
    n, c, h, w = imgs.shape
    k = wT.shape[0]
    hw = h * w
    hwp = -(-hw // 128) * 128
    x = imgs.reshape(n, c, hw).astype(jnp.bfloat16)
    gt = heatmaps.reshape(n, k, hw).astype(jnp.bfloat16)
    if hwp != hw:
        x = jnp.pad(x, ((0, 0), (0, 0), (0, hwp - hw)))
        gt = jnp.pad(gt, ((0, 0), (0, 0), (0, hwp - hw)))
    pred, lpart = _fused_call(x, wT.astype(jnp.bfloat16), b, gt, hw)
    if hwp != hw:
        pred = pred[:, :, :hw]
    loss = jnp.sum(lpart[:, 0, 0]) * (1.0 / float(heatmaps.size))
    return [pred.astype(jnp.float32).reshape(n, k, h, w), loss]


# speedup vs baseline: 6.8572x; 3.5930x over previous
import jax
import jax.numpy as jnp
from jax.experimental import pallas as pl
from jax.experimental.pallas import tpu as pltpu


def _fused_mse_kernel(x_ref, w_ref, b_ref, gt_ref, pred_ref, lpart_ref):
    h, w_sp, c = x_ref.shape[1:]
    k = w_ref.shape[0]
    x2 = x_ref[0].reshape(h * w_sp, c)
    pred = jax.lax.dot_general(
        x2, w_ref[...], (((1,), (1,)), ((), ())),
        preferred_element_type=jnp.float32)
    pred = pred + b_ref[...]
    pred_ref[0] = pred.reshape(h, w_sp, k)
    d = pred - gt_ref[0].reshape(h * w_sp, k)
    lpart_ref[...] = jnp.broadcast_to(jnp.sum(d * d), lpart_ref.shape)


def kernel(imgs, heatmaps, extra, wT, b):
    n, c, h, w = imgs.shape
    k = wT.shape[0]
    x_t = jnp.transpose(imgs, (0, 2, 3, 1))
    gt_t = jnp.transpose(heatmaps, (0, 2, 3, 1))
    b_row = b.reshape(1, k)
    pred_t, lpart = pl.pallas_call(
        _fused_mse_kernel,
        out_shape=(jax.ShapeDtypeStruct((n, h, w, k), jnp.float32),
                   jax.ShapeDtypeStruct((n, 1, 128), jnp.float32)),
        grid=(n,),
        in_specs=[
            pl.BlockSpec((1, h, w, c), lambda i: (i, 0, 0, 0)),
            pl.BlockSpec((k, c), lambda i: (0, 0)),
            pl.BlockSpec((1, k), lambda i: (0, 0)),
            pl.BlockSpec((1, h, w, k), lambda i: (i, 0, 0, 0)),
        ],
        out_specs=(
            pl.BlockSpec((1, h, w, k), lambda i: (i, 0, 0, 0)),
            pl.BlockSpec((1, 1, 128), lambda i: (i, 0, 0)),
        ),
        compiler_params=pltpu.CompilerParams(
            dimension_semantics=("parallel",)),
    )(x_t, wT, b_row, gt_t)
    loss = jnp.sum(lpart[:, 0, 0]) * (1.0 / float(heatmaps.size))
    return [jnp.transpose(pred_t, (0, 3, 1, 2)), loss]


# speedup vs baseline: 7.2725x; 1.0606x over previous
import jax
import jax.numpy as jnp
from jax.experimental import pallas as pl
from jax.experimental.pallas import tpu as pltpu


def _fused_mse_kernel(x_ref, w_ref, b_ref, gt_ref, pred_ref, lpart_ref):
    nb, h, w_sp, c = x_ref.shape
    k = w_ref.shape[0]
    x2 = x_ref[...].reshape(nb * h * w_sp, c)
    pred = jax.lax.dot_general(
        x2, w_ref[...], (((1,), (1,)), ((), ())),
        preferred_element_type=jnp.float32)
    pred = pred + b_ref[...]
    pred_ref[...] = pred.reshape(nb, h, w_sp, k)
    d = pred - gt_ref[...].reshape(nb * h * w_sp, k)
    lpart_ref[...] = jnp.broadcast_to(jnp.sum(d * d), lpart_ref.shape)


def kernel(imgs, heatmaps, extra, wT, b):
    n, c, h, w = imgs.shape
    k = wT.shape[0]
    nb = 2 if n % 2 == 0 else 1
    x_t = jnp.transpose(imgs, (0, 2, 3, 1))
    gt_t = jnp.transpose(heatmaps, (0, 2, 3, 1))
    b_row = b.reshape(1, k)
    pred_t, lpart = pl.pallas_call(
        _fused_mse_kernel,
        out_shape=(jax.ShapeDtypeStruct((n, h, w, k), jnp.float32),
                   jax.ShapeDtypeStruct((n // nb, 1, 128), jnp.float32)),
        grid=(n // nb,),
        in_specs=[
            pl.BlockSpec((nb, h, w, c), lambda i: (i, 0, 0, 0)),
            pl.BlockSpec((k, c), lambda i: (0, 0)),
            pl.BlockSpec((1, k), lambda i: (0, 0)),
            pl.BlockSpec((nb, h, w, k), lambda i: (i, 0, 0, 0)),
        ],
        out_specs=(
            pl.BlockSpec((nb, h, w, k), lambda i: (i, 0, 0, 0)),
            pl.BlockSpec((1, 1, 128), lambda i: (i, 0, 0)),
        ),
        compiler_params=pltpu.CompilerParams(
            dimension_semantics=("parallel",)),
    )(x_t, wT, b_row, gt_t)
    loss = jnp.sum(lpart[:, 0, 0]) * (1.0 / float(heatmaps.size))
    return [jnp.transpose(pred_t, (0, 3, 1, 2)), loss]


# speedup vs baseline: 7.3703x; 1.0134x over previous
import jax
import jax.numpy as jnp
from jax.experimental import pallas as pl
from jax.experimental.pallas import tpu as pltpu


def _fused_mse_kernel(x_ref, w_ref, b_ref, gt_ref, pred_ref, lpart_ref):
    nb, h, w_sp, c = x_ref.shape
    k = w_ref.shape[0]
    x2 = x_ref[...].reshape(nb * h * w_sp, c)
    pred = jax.lax.dot_general(
        x2, w_ref[...], (((1,), (1,)), ((), ())),
        preferred_element_type=jnp.float32)
    pred = pred + b_ref[...]
    pred_ref[...] = pred.reshape(nb, h, w_sp, k)
    d = pred - gt_ref[...].reshape(nb * h * w_sp, k)
    lpart_ref[...] = jnp.broadcast_to(jnp.sum(d * d), lpart_ref.shape)


def kernel(imgs, heatmaps, extra, wT, b):
    n, c, h, w = imgs.shape
    k = wT.shape[0]
    nb = 4 if n % 4 == 0 else (2 if n % 2 == 0 else 1)
    x_t = jnp.transpose(imgs, (0, 2, 3, 1))
    gt_t = jnp.transpose(heatmaps, (0, 2, 3, 1))
    b_row = b.reshape(1, k)
    pred_t, lpart = pl.pallas_call(
        _fused_mse_kernel,
        out_shape=(jax.ShapeDtypeStruct((n, h, w, k), jnp.float32),
                   jax.ShapeDtypeStruct((n // nb, 1, 128), jnp.float32)),
        grid=(n // nb,),
        in_specs=[
            pl.BlockSpec((nb, h, w, c), lambda i: (i, 0, 0, 0)),
            pl.BlockSpec((k, c), lambda i: (0, 0)),
            pl.BlockSpec((1, k), lambda i: (0, 0)),
            pl.BlockSpec((nb, h, w, k), lambda i: (i, 0, 0, 0)),
        ],
        out_specs=(
            pl.BlockSpec((nb, h, w, k), lambda i: (i, 0, 0, 0)),
            pl.BlockSpec((1, 1, 128), lambda i: (i, 0, 0)),
        ),
        compiler_params=pltpu.CompilerParams(
            dimension_semantics=("parallel",)),
    )(x_t, wT, b_row, gt_t)
    loss = jnp.sum(lpart[:, 0, 0]) * (1.0 / float(heatmaps.size))
    return [jnp.transpose(pred_t, (0, 3, 1, 2)), loss]
